# baseline (speedup 1.0000x reference)
import functools
import jax
import jax.numpy as jnp
from jax import lax
from jax.experimental import pallas as pl
from jax.experimental.pallas import tpu as pltpu
from jax.experimental.pallas import tpu_sc as plsc

B, C = 1024, 7
CC = C * C
NW = 32
CHUNKS = CC * 128
PER_W = CHUNKS // NW
XPAD = B * C + 128

_mesh = plsc.VectorSubcoreMesh(core_axis_name="c", subcore_axis_name="s")


def _full(v):
    return jnp.full((16,), v, jnp.int32)


@functools.partial(
    pl.kernel, mesh=_mesh,
    compiler_params=pltpu.CompilerParams(needs_layout_passes=False),
    out_type=jax.ShapeDtypeStruct((CHUNKS * 8, 8, 128), jnp.float32),
    scratch_types=[
        pltpu.VMEM((XPAD,), jnp.float32),
        pltpu.VMEM((16,), jnp.float32),
        pltpu.VMEM((XPAD,), jnp.int32),
        pltpu.VMEM((16 * B,), jnp.float32),
        pltpu.VMEM((8, 8, 128), jnp.float32),
        pltpu.VMEM((8, 8, 128), jnp.float32),
        pltpu.SemaphoreType.DMA,
        pltpu.SemaphoreType.DMA,
    ],
)
def _sc_body(x_hbm, w_hbm, out_hbm, xv, wv, idxv, abf, cb0, cb1,
             sem0, sem1):
    wid = lax.axis_index("s") * 2 + lax.axis_index("c")
    pltpu.sync_copy(x_hbm, xv)
    pltpu.sync_copy(w_hbm, wv)

    lanes = lax.iota(jnp.int32, 16)

    def mk_idx(g, carry):
        v = xv[pl.ds(g * 16, 16)]
        idxv[pl.ds(g * 16, 16)] = (v * (v >= 0).astype(v.dtype)
                                   ).astype(jnp.int32)
        return carry
    lax.fori_loop(0, XPAD // 16, mk_idx, 0)

    def mk_ab(g, carry):
        k = g // (B // 16)
        jg = g % (B // 16)
        jidx = lanes + _full(jg * 16)
        vals = plsc.load_gather(xv, [jidx * _full(C) + _full(k)])
        wk = plsc.load_gather(wv, [_full(k)])
        abf[pl.ds(k * B + jg * 16, 16)] = jnp.abs(vals) * wk
        abf[pl.ds((8 + k) * B + jg * 16, 16)] = jnp.abs(vals - 1.0) * wk
        return carry
    lax.fori_loop(0, C * (B // 16), mk_ab, 0)

    base = wid * PER_W

    def fill(buf, g):
        ck = g // 128
        t = g % 128
        c = ck // C
        k = ck % C
        kvec = _full(k)
        sels = plsc.load_gather(
            idxv, [(_full(t * 8) + lanes) * _full(C) + _full(c)])
        srcv = kvec + _full(8) * (sels == kvec).astype(jnp.int32)
        for r in range(8):
            rowbase = srcv[r] * B

            def seg(jt, carry2, _r=r, _rowbase=rowbase):
                cbase = _full(_rowbase + jt * 128) + lanes
                for gg in range(8):
                    buf[jt, _r, pl.ds(gg * 16, 16)] = plsc.load_gather(
                        abf, [cbase + _full(gg * 16)])
                return carry2
            lax.fori_loop(0, 8, seg, 0)

    def start(buf, sem, g):
        pltpu.make_async_copy(buf, out_hbm.at[pl.ds(g * 8, 8)], sem).start()

    bufs = (cb0, cb1)
    sems = (sem0, sem1)
    for b in range(2):
        fill(bufs[b], base + b)
        start(bufs[b], sems[b], base + b)

    def pair(p, carry):
        for b in range(2):
            g = base + 2 + p * 2 + b
            pltpu.make_async_copy(
                bufs[b], out_hbm.at[pl.ds((g - 2) * 8, 8)], sems[b]).wait()
            fill(bufs[b], g)
            start(bufs[b], sems[b], g)
        return carry
    lax.fori_loop(0, (PER_W - 2) // 2, pair, 0)

    for b in range(2):
        g_last = base + PER_W - 2 + b
        pltpu.make_async_copy(
            bufs[b], out_hbm.at[pl.ds(g_last * 8, 8)], sems[b]).wait()


def kernel(input, target, code_weights):
    x = input.reshape(B, C)
    xpad = jnp.zeros((XPAD,), jnp.float32).at[:B * C].set(x.reshape(-1))
    wpad = jnp.zeros((16,), jnp.float32).at[:C].set(code_weights)
    out3 = _sc_body(xpad, wpad)
    out6 = out3.reshape(C, C, 128, 8, 8, 128)
    return out6.transpose(2, 4, 3, 5, 0, 1).reshape(B, B, C, C)


# speedup vs baseline: 4.5511x; 4.5511x over previous
import functools
import jax
import jax.numpy as jnp
from jax import lax
from jax.experimental import pallas as pl
from jax.experimental.pallas import tpu as pltpu
from jax.experimental.pallas import tpu_sc as plsc

B, C = 1024, 7
CC = C * C
NW = 32
ROWS_W = B // NW
XPAD = B * C + 128

_mesh = plsc.VectorSubcoreMesh(core_axis_name="c", subcore_axis_name="s")


def _full(v):
    return jnp.full((16,), v, jnp.int32)


@functools.partial(
    pl.kernel, mesh=_mesh,
    compiler_params=pltpu.CompilerParams(needs_layout_passes=False),
    out_type=jax.ShapeDtypeStruct((B // 8, 8, 128), jnp.float32),
    scratch_types=[
        pltpu.VMEM((XPAD,), jnp.float32),
        pltpu.VMEM((XPAD,), jnp.int32),
        pltpu.VMEM((ROWS_W // 8, 8, 128), jnp.float32),
        pltpu.SemaphoreType.DMA,
    ],
)
def _sc_onehot(x_hbm, oh_hbm, xv, idxv, obuf, sem):
    wid = lax.axis_index("s") * 2 + lax.axis_index("c")
    pltpu.sync_copy(x_hbm, xv)
    lanes = lax.iota(jnp.int32, 16)

    def mk_idx(g, carry):
        v = xv[pl.ds(g * 16, 16)]
        idxv[pl.ds(g * 16, 16)] = (v * (v >= 0).astype(v.dtype)
                                   ).astype(jnp.int32)
        return carry
    lax.fori_loop(0, XPAD // 16, mk_idx, 0)

    base_i = wid * ROWS_W
    for t in range(ROWS_W // 8):
        for r in range(8):
            i = base_i + t * 8 + r
            for g in range(8):
                lvec = lanes + _full(g * 16)
                cvec = lvec // _full(C)
                kvec = lvec % _full(C)
                sel = plsc.load_gather(idxv, [_full(i * C) + cvec])
                val = ((sel == kvec) & (lvec < _full(CC))
                       ).astype(jnp.float32)
                obuf[t, r, pl.ds(g * 16, 16)] = val
    pltpu.make_async_copy(
        obuf, oh_hbm.at[pl.ds(wid * (ROWS_W // 8), ROWS_W // 8)], sem).start()
    pltpu.make_async_copy(
        obuf, oh_hbm.at[pl.ds(wid * (ROWS_W // 8), ROWS_W // 8)], sem).wait()


def _tc_body(w_ref, xT_ref, oh_ref, out_ref):
    c = pl.program_id(0)
    k = pl.program_id(1)
    ck = c * C + k
    ohb = oh_ref[...].reshape(B, 128).astype(jnp.bfloat16)
    sel = (jax.lax.broadcasted_iota(jnp.int32, (128, B), 0) == ck
           ).astype(jnp.bfloat16)
    m = jax.lax.dot_general(
        ohb, sel,
        dimension_numbers=(((1,), (0,)), ((), ())),
        preferred_element_type=jnp.float32,
    )
    xk = xT_ref[...].reshape(1, B)
    wk = w_ref[k]
    out_ref[...] = (jnp.abs(xk - m) * wk).reshape(1, 1, B, B)


def kernel(input, target, code_weights):
    x = input.reshape(B, C)
    xpad = jnp.zeros((XPAD,), jnp.float32).at[:B * C].set(x.reshape(-1))
    oh3 = _sc_onehot(xpad)
    xT = x.T.reshape(C, 1, B)

    out = pl.pallas_call(
        _tc_body,
        grid=(C, C),
        in_specs=[
            pl.BlockSpec(memory_space=pltpu.SMEM),
            pl.BlockSpec((1, 1, B), lambda c, k: (k, 0, 0)),
            pl.BlockSpec((B // 8, 8, 128), lambda c, k: (0, 0, 0)),
        ],
        out_specs=pl.BlockSpec((1, 1, B, B), lambda c, k: (c, k, 0, 0)),
        out_shape=jax.ShapeDtypeStruct((C, C, B, B), jnp.float32),
    )(code_weights, xT, oh3)
    return out.transpose(2, 3, 0, 1)


# speedup vs baseline: 6.1271x; 1.3463x over previous
import jax
import jax.numpy as jnp
from jax.experimental import pallas as pl
from jax.experimental.pallas import tpu as pltpu

B, C = 1024, 7
CC = C * C


BI = 1024


def _body(w_ref, xT_ref, xrep_ref, out_ref, oh_ref):
    c = pl.program_id(0)
    k = pl.program_id(1)
    ib = pl.program_id(2)

    @pl.when((c == 0) & (k == 0) & (ib == 0))
    def _():
        xi = xrep_ref[...]
        idx = (xi * (xi >= 0).astype(xi.dtype)).astype(jnp.int32)
        lio = jax.lax.broadcasted_iota(jnp.int32, (B, CC), 1)
        oh_ref[...] = (idx == lio % C).astype(jnp.bfloat16)

    ck = c * C + k
    sel = (jax.lax.broadcasted_iota(jnp.int32, (CC, B), 0) == ck
           ).astype(jnp.bfloat16)
    m = jax.lax.dot_general(
        oh_ref[pl.ds(ib * BI, BI), :], sel,
        dimension_numbers=(((1,), (0,)), ((), ())),
        preferred_element_type=jnp.float32,
    )
    xk = xT_ref[...].reshape(1, B)
    wk = w_ref[k]
    out_ref[...] = (jnp.abs(xk - m) * wk).reshape(1, 1, BI, B)


def kernel(input, target, code_weights):
    x = input.reshape(B, C)
    xT = x.T.reshape(C, 1, B)
    xrep = jnp.repeat(x, C, axis=1)

    out = pl.pallas_call(
        _body,
        grid=(C, C, B // BI),
        in_specs=[
            pl.BlockSpec(memory_space=pltpu.SMEM),
            pl.BlockSpec((1, 1, B), lambda c, k, ib: (k, 0, 0)),
            pl.BlockSpec((B, CC), lambda c, k, ib: (0, 0)),
        ],
        out_specs=pl.BlockSpec((1, 1, BI, B), lambda c, k, ib: (c, k, ib, 0)),
        out_shape=jax.ShapeDtypeStruct((C, C, B, B), jnp.float32),
        scratch_shapes=[pltpu.VMEM((B, CC), jnp.bfloat16)],
    )(code_weights, xT, xrep)
    return out.transpose(2, 3, 0, 1)
